# Initial kernel scaffold; baseline (speedup 1.0000x reference)
import jax
import jax.numpy as jnp
from jax.experimental import pallas as pl


def kernel(p_i, mos, p_j, error_p_i_flow, nearest_flow):
    raise NotImplementedError("write your pallas kernel here")


# speedup vs baseline: 1.4244x; 1.4244x over previous
import jax
import jax.numpy as jnp
from jax import lax
from jax.experimental import pallas as pl
from jax.experimental.pallas import tpu as pltpu
from jax.experimental.pallas import tpu_sc as plsc

_G = 320
_GG = _G * _G
_X_MIN = -35.0
_Y_MIN = -35.0
_CELL = abs(2.0 * _X_MIN / _G)
_TI = 256


def _colmask(t, nt):
    return lax.broadcasted_iota(jnp.int32, (_TI, nt), 1) == t


def _getcol(block, mask):
    return jnp.sum(jnp.where(mask, block, jnp.zeros_like(block)),
                   axis=1, keepdims=True)


def _knn_body(xiT, yiT, ziT, xj, yj, zj, dxT, ixT, dyo):
    N = xj.shape[2]
    nt = xiT.shape[2]
    t = pl.program_id(1)
    cm = _colmask(t, nt)
    d = jnp.abs(_getcol(xiT[0], cm) - xj[0])
    d = d + jnp.abs(_getcol(yiT[0], cm) - yj[0])
    d = d + jnp.abs(_getcol(ziT[0], cm) - zj[0])
    rmin = jnp.min(d, axis=1, keepdims=True)
    jj = lax.broadcasted_iota(jnp.int32, (_TI, N), 1)
    amin = jnp.min(jnp.where(d == rmin, jj, N), axis=1, keepdims=True)
    dxT[0] = jnp.where(cm, rmin, dxT[0])
    ixT[0] = jnp.where(cm, amin, ixT[0])
    cmin = jnp.min(d, axis=0, keepdims=True)

    @pl.when(t == 0)
    def _():
        dyo[0] = cmin

    @pl.when(t != 0)
    def _():
        dyo[0] = jnp.minimum(dyo[0], cmin)


def _knn_call(xiT, yiT, ziT, xj, yj, zj):
    B, _, N = xj.shape
    nt = N // _TI
    row = pl.BlockSpec((1, 1, N), lambda b, t: (b, 0, 0))
    col = pl.BlockSpec((1, _TI, nt), lambda b, t: (b, 0, 0))
    return pl.pallas_call(
        _knn_body,
        grid=(B, nt),
        in_specs=[col, col, col, row, row, row],
        out_specs=[col, col, row],
        out_shape=[
            jax.ShapeDtypeStruct((B, _TI, nt), jnp.float32),
            jax.ShapeDtypeStruct((B, _TI, nt), jnp.int32),
            jax.ShapeDtypeStruct((B, 1, N), jnp.float32),
        ],
        compiler_params=pltpu.CompilerParams(
            dimension_semantics=("arbitrary", "arbitrary"),
        ),
    )(xiT, yiT, ziT, xj, yj, zj)


def _winner_body(cellT, labT, m0T, m1T, cell, out, psum, pcnt):
    N = cell.shape[2]
    b = pl.program_id(0)
    t = pl.program_id(1)
    nb = pl.num_programs(0)
    nt = pl.num_programs(1)

    @pl.when((b == 0) & (t == 0))
    def _():
        psum[...] = jnp.zeros_like(psum)
        pcnt[...] = jnp.zeros_like(pcnt)

    rowc = cell[0]
    cm = _colmask(t, cellT.shape[2])
    jj = lax.broadcasted_iota(jnp.int32, (_TI, N), 1)
    ii = lax.broadcasted_iota(jnp.int32, (_TI, 1), 0) + t * _TI
    ci = _getcol(cellT[0], cm)
    hit = (ci == rowc) & (jj > ii)
    later = jnp.any(hit, axis=1, keepdims=True)
    win = (ci != -1) & jnp.logical_not(later)
    m0i = _getcol(m0T[0], cm)
    m1i = _getcol(m1T[0], cm)
    labi = _getcol(labT[0], cm)
    mx = jnp.maximum(m0i, m1i)
    lse = jnp.log(jnp.exp(m0i - mx) + jnp.exp(m1i - mx)) + mx
    sel = jnp.where(labi == 1, m1i, m0i)
    psum[...] = psum[...] + jnp.sum(jnp.where(win, sel - lse, 0.0))
    pcnt[...] = pcnt[...] + jnp.sum(win.astype(jnp.float32))

    @pl.when((b == nb - 1) & (t == nt - 1))
    def _():
        out[...] = -(psum[...] / pcnt[...])


def _winner_call(cellT, labT, m0T, m1T, cell):
    B, _, N = cell.shape
    nt = N // _TI
    row = pl.BlockSpec((1, 1, N), lambda b, t: (b, 0, 0))
    col = pl.BlockSpec((1, _TI, nt), lambda b, t: (b, 0, 0))
    return pl.pallas_call(
        _winner_body,
        grid=(B, nt),
        in_specs=[col, col, col, col, row],
        out_specs=pl.BlockSpec((1, 1), lambda b, t: (0, 0)),
        out_shape=jax.ShapeDtypeStruct((1, 1), jnp.float32),
        scratch_shapes=[
            pltpu.VMEM((1, 1), jnp.float32),
            pltpu.VMEM((1, 1), jnp.float32),
        ],
        compiler_params=pltpu.CompilerParams(
            dimension_semantics=("arbitrary", "arbitrary"),
        ),
    )(cellT, labT, m0T, m1T, cell)


def _sc_stage(pjx, pjy, dx, dy, ef, ix, nf, mos0, mos1):
    B, N = pjx.shape
    npt = B * N
    nw = 32
    ppw = npt // nw
    wpb = N // ppw

    mesh = plsc.VectorSubcoreMesh(core_axis_name="c", subcore_axis_name="s")

    def _sc_body(pjx_h, pjy_h, dx_h, dy_h, ef_h, ix_h, nf_h, mos0_h, mos1_h,
                 cell_o, lab_o, m0_o, m1_o,
                 pjx_v, pjy_v, dx_v, dy_v, ef_v, ix_v, nf_v,
                 cell_v, lab_v, m0_v, m1_v, plane_v, sem):
        c = lax.axis_index("c")
        s = lax.axis_index("s")
        b = (c * 16 + s) // wpb
        off = ((c * 16 + s) % wpb) * ppw
        r = c * 16 + s
        pltpu.sync_copy(pjx_h.at[b], pjx_v)
        pltpu.sync_copy(pjy_h.at[b], pjy_v)
        pltpu.sync_copy(dx_h.at[b, pl.ds(off, ppw)], dx_v)
        pltpu.sync_copy(dy_h.at[b, pl.ds(off, ppw)], dy_v)
        pltpu.sync_copy(ef_h.at[b, pl.ds(off, ppw)], ef_v)
        pltpu.sync_copy(ix_h.at[b, pl.ds(off, ppw)], ix_v)
        pltpu.sync_copy(nf_h.at[b, pl.ds(off, ppw)], nf_v)
        one16 = jnp.full((16,), 1, jnp.int32)
        zero16 = jnp.full((16,), 0, jnp.int32)
        neg16 = jnp.full((16,), -1, jnp.int32)
        for j in range(ppw // 16):
            sl = pl.ds(j * 16, 16)
            err = (dx_v[sl] + dy_v[sl]) / 2.0
            dyn = ef_v[sl] > err
            idxv = jnp.where(dyn, nf_v[sl], ix_v[sl])
            labv = jnp.where(dyn, one16, zero16)
            xjv = plsc.load_gather(pjx_v, [idxv])
            yjv = plsc.load_gather(pjy_v, [idxv])
            cxi = ((xjv - _X_MIN) / _CELL).astype(jnp.int32)
            cyi = ((yjv - _Y_MIN) / _CELL).astype(jnp.int32)
            okv = (cxi >= 0) & (cxi < _G) & (cyi >= 0) & (cyi < _G)
            cellv = jnp.where(okv, cxi * _G + cyi, neg16)
            cell_v[sl] = cellv
            lab_v[sl] = labv
        pltpu.sync_copy(mos0_h.at[b], plane_v)
        for j in range(ppw // 16):
            sl = pl.ds(j * 16, 16)
            cellv = cell_v[sl]
            gidx = jnp.where(cellv < 0, jnp.full((16,), 0, jnp.int32), cellv)
            m0_v[sl] = plsc.load_gather(plane_v, [gidx])
        pltpu.sync_copy(mos1_h.at[b], plane_v)
        for j in range(ppw // 16):
            sl = pl.ds(j * 16, 16)
            cellv = cell_v[sl]
            gidx = jnp.where(cellv < 0, jnp.full((16,), 0, jnp.int32), cellv)
            m1_v[sl] = plsc.load_gather(plane_v, [gidx])
        pltpu.sync_copy(cell_v, cell_o.at[r])
        pltpu.sync_copy(lab_v, lab_o.at[r])
        pltpu.sync_copy(m0_v, m0_o.at[r])
        pltpu.sync_copy(m1_v, m1_o.at[r])

    f = pl.kernel(
        _sc_body,
        out_type=(
            jax.ShapeDtypeStruct((nw, ppw), jnp.int32),
            jax.ShapeDtypeStruct((nw, ppw), jnp.int32),
            jax.ShapeDtypeStruct((nw, ppw), jnp.float32),
            jax.ShapeDtypeStruct((nw, ppw), jnp.float32),
        ),
        mesh=mesh,
        compiler_params=pltpu.CompilerParams(
            needs_layout_passes=False, use_tc_tiling_on_sc=False),
        scratch_types=(
            pltpu.VMEM((N,), jnp.float32),
            pltpu.VMEM((N,), jnp.float32),
            pltpu.VMEM((ppw,), jnp.float32),
            pltpu.VMEM((ppw,), jnp.float32),
            pltpu.VMEM((ppw,), jnp.float32),
            pltpu.VMEM((ppw,), jnp.int32),
            pltpu.VMEM((ppw,), jnp.int32),
            pltpu.VMEM((ppw,), jnp.int32),
            pltpu.VMEM((ppw,), jnp.int32),
            pltpu.VMEM((ppw,), jnp.float32),
            pltpu.VMEM((ppw,), jnp.float32),
            pltpu.VMEM((_GG,), jnp.float32),
            pltpu.SemaphoreType.DMA,
        ),
    )
    cell_o, lab_o, m0_o, m1_o = f(pjx, pjy, dx, dy, ef, ix, nf, mos0, mos1)
    return (cell_o.reshape(B, N), lab_o.reshape(B, N),
            m0_o.reshape(B, N), m1_o.reshape(B, N))


def kernel(p_i, mos, p_j, error_p_i_flow, nearest_flow):
    B, N, _ = p_i.shape
    nt = N // _TI
    xi, yi, zi = p_i[:, :, 0], p_i[:, :, 1], p_i[:, :, 2]
    xj, yj, zj = p_j[:, :, 0], p_j[:, :, 1], p_j[:, :, 2]

    def to_col(a):
        return a.reshape(B, nt, _TI).transpose(0, 2, 1)

    def to_row(a):
        return a.reshape(B, 1, N)

    dxT, ixT, dy = _knn_call(
        to_col(xi), to_col(yi), to_col(zi), to_row(xj), to_row(yj), to_row(zj))
    dx = dxT.transpose(0, 2, 1).reshape(B, N)
    ix = ixT.transpose(0, 2, 1).reshape(B, N)
    dy = dy.reshape(B, N)

    cell, lab, m0, m1 = _sc_stage(
        xj, yj, dx, dy, error_p_i_flow, ix, nearest_flow[..., 0],
        mos[:, 0].reshape(B, _GG), mos[:, 1].reshape(B, _GG))

    loss = _winner_call(
        to_col(cell), to_col(lab), to_col(m0), to_col(m1), to_row(cell))
    return loss[0, 0]


# speedup vs baseline: 1.4367x; 1.0087x over previous
import jax
import jax.numpy as jnp
from jax import lax
from jax.experimental import pallas as pl
from jax.experimental.pallas import tpu as pltpu
from jax.experimental.pallas import tpu_sc as plsc

_G = 320
_GG = _G * _G
_X_MIN = -35.0
_Y_MIN = -35.0
_CELL = abs(2.0 * _X_MIN / _G)
_TI = 256

_DN_COL = (((1,), (1,)), ((), ()))
_DN_ROW = (((0,), (0,)), ((), ()))


def _eye():
    return (lax.broadcasted_iota(jnp.int32, (_TI, _TI), 0)
            == lax.broadcasted_iota(jnp.int32, (_TI, _TI), 1)
            ).astype(jnp.float32)


def _to_col(eye, seg):
    return lax.dot_general(eye, seg, _DN_COL,
                           preferred_element_type=jnp.float32,
                           precision=lax.Precision.HIGHEST)


def _to_row(col, eye):
    return lax.dot_general(col, eye, _DN_ROW,
                           preferred_element_type=jnp.float32,
                           precision=lax.Precision.HIGHEST)


def _knn_body(piT, pjT, dxo, ixo, dyo):
    N = pjT.shape[2]
    t = pl.program_id(1)
    off = pl.multiple_of(t * _TI, _TI)
    eye = _eye()
    isl = pl.ds(off, _TI)
    xit = _to_col(eye, piT[0, pl.ds(0, 1), isl])
    yit = _to_col(eye, piT[0, pl.ds(1, 1), isl])
    zit = _to_col(eye, piT[0, pl.ds(2, 1), isl])
    d = jnp.abs(xit - pjT[0, pl.ds(0, 1), :])
    d = d + jnp.abs(yit - pjT[0, pl.ds(1, 1), :])
    d = d + jnp.abs(zit - pjT[0, pl.ds(2, 1), :])
    rmin = jnp.min(d, axis=1, keepdims=True)
    jj = lax.broadcasted_iota(jnp.int32, (_TI, N), 1)
    amin = jnp.min(jnp.where(d == rmin, jj, N), axis=1, keepdims=True)
    dxo[0, :, isl] = _to_row(rmin, eye)
    ixo[0, :, isl] = _to_row(amin.astype(jnp.float32), eye).astype(jnp.int32)
    cmin = jnp.min(d, axis=0, keepdims=True)

    @pl.when(t == 0)
    def _():
        dyo[0] = cmin

    @pl.when(t != 0)
    def _():
        dyo[0] = jnp.minimum(dyo[0], cmin)


def _knn_call(piT, pjT):
    B, _, N = pjT.shape
    nt = N // _TI
    coords = pl.BlockSpec((1, 3, N), lambda b, t: (b, 0, 0))
    row = pl.BlockSpec((1, 1, N), lambda b, t: (b, 0, 0))
    return pl.pallas_call(
        _knn_body,
        grid=(B, nt),
        in_specs=[coords, coords],
        out_specs=[row, row, row],
        out_shape=[
            jax.ShapeDtypeStruct((B, 1, N), jnp.float32),
            jax.ShapeDtypeStruct((B, 1, N), jnp.int32),
            jax.ShapeDtypeStruct((B, 1, N), jnp.float32),
        ],
        compiler_params=pltpu.CompilerParams(
            dimension_semantics=("arbitrary", "arbitrary"),
        ),
    )(piT, pjT)


def _winner_body(cell, lab, m0, m1, out, psum, pcnt):
    N = cell.shape[2]
    b = pl.program_id(0)
    t = pl.program_id(1)
    nb = pl.num_programs(0)
    nt = pl.num_programs(1)
    off = pl.multiple_of(t * _TI, _TI)
    isl = pl.ds(off, _TI)
    eye = _eye()

    @pl.when((b == 0) & (t == 0))
    def _():
        psum[...] = jnp.zeros_like(psum)
        pcnt[...] = jnp.zeros_like(pcnt)

    rowcf = cell[0].astype(jnp.float32)
    cif = _to_col(eye, cell[0, :, isl].astype(jnp.float32))
    jj = lax.broadcasted_iota(jnp.int32, (_TI, N), 1)
    ii = lax.broadcasted_iota(jnp.int32, (_TI, 1), 0) + t * _TI
    hit = (cif == rowcf) & (jj > ii)
    later = jnp.any(hit, axis=1, keepdims=True)
    win = (cif != -1.0) & jnp.logical_not(later)
    m0i = _to_col(eye, m0[0, :, isl])
    m1i = _to_col(eye, m1[0, :, isl])
    labi = _to_col(eye, lab[0, :, isl].astype(jnp.float32))
    mx = jnp.maximum(m0i, m1i)
    lse = jnp.log(jnp.exp(m0i - mx) + jnp.exp(m1i - mx)) + mx
    sel = jnp.where(labi == 1.0, m1i, m0i)
    psum[...] = psum[...] + jnp.sum(jnp.where(win, sel - lse, 0.0))
    pcnt[...] = pcnt[...] + jnp.sum(win.astype(jnp.float32))

    @pl.when((b == nb - 1) & (t == nt - 1))
    def _():
        out[...] = -(psum[...] / pcnt[...])


def _winner_call(cell, lab, m0, m1):
    B, _, N = cell.shape
    nt = N // _TI
    row = pl.BlockSpec((1, 1, N), lambda b, t: (b, 0, 0))
    return pl.pallas_call(
        _winner_body,
        grid=(B, nt),
        in_specs=[row, row, row, row],
        out_specs=pl.BlockSpec((1, 1), lambda b, t: (0, 0)),
        out_shape=jax.ShapeDtypeStruct((1, 1), jnp.float32),
        scratch_shapes=[
            pltpu.VMEM((1, 1), jnp.float32),
            pltpu.VMEM((1, 1), jnp.float32),
        ],
        compiler_params=pltpu.CompilerParams(
            dimension_semantics=("arbitrary", "arbitrary"),
        ),
    )(cell, lab, m0, m1)


def _sc_stage(pjT, dx, dy, ef, ix, nf, mos2):
    B, _, N = pjT.shape
    npt = B * N
    nw = 32
    ppw = npt // nw
    wpb = N // ppw

    mesh = plsc.VectorSubcoreMesh(core_axis_name="c", subcore_axis_name="s")

    def _sc_body(pjT_h, dx_h, dy_h, ef_h, ix_h, nf_h, mos_h,
                 cell_o, lab_o, m0_o, m1_o,
                 pjx_v, pjy_v, dx_v, dy_v, ef_v, ix_v, nf_v,
                 cell_v, lab_v, m0_v, m1_v, plane_v, sem):
        c = lax.axis_index("c")
        s = lax.axis_index("s")
        b = (c * 16 + s) // wpb
        off = ((c * 16 + s) % wpb) * ppw
        r = c * 16 + s
        pltpu.sync_copy(pjT_h.at[b, 0], pjx_v)
        pltpu.sync_copy(pjT_h.at[b, 1], pjy_v)
        pltpu.sync_copy(dx_h.at[b, 0, pl.ds(off, ppw)], dx_v)
        pltpu.sync_copy(dy_h.at[b, 0, pl.ds(off, ppw)], dy_v)
        pltpu.sync_copy(ef_h.at[b, pl.ds(off, ppw)], ef_v)
        pltpu.sync_copy(ix_h.at[b, 0, pl.ds(off, ppw)], ix_v)
        pltpu.sync_copy(nf_h.at[b, pl.ds(off, ppw)], nf_v)
        one16 = jnp.full((16,), 1, jnp.int32)
        zero16 = jnp.full((16,), 0, jnp.int32)
        neg16 = jnp.full((16,), -1, jnp.int32)
        for j in range(ppw // 16):
            sl = pl.ds(j * 16, 16)
            err = (dx_v[sl] + dy_v[sl]) / 2.0
            dyn = ef_v[sl] > err
            idxv = jnp.where(dyn, nf_v[sl], ix_v[sl])
            labv = jnp.where(dyn, one16, zero16)
            xjv = plsc.load_gather(pjx_v, [idxv])
            yjv = plsc.load_gather(pjy_v, [idxv])
            cxi = ((xjv - _X_MIN) / _CELL).astype(jnp.int32)
            cyi = ((yjv - _Y_MIN) / _CELL).astype(jnp.int32)
            okv = (cxi >= 0) & (cxi < _G) & (cyi >= 0) & (cyi < _G)
            cellv = jnp.where(okv, cxi * _G + cyi, neg16)
            cell_v[sl] = cellv
            lab_v[sl] = labv
        pltpu.sync_copy(mos_h.at[b, 0], plane_v)
        for j in range(ppw // 16):
            sl = pl.ds(j * 16, 16)
            cellv = cell_v[sl]
            gidx = jnp.where(cellv < 0, jnp.full((16,), 0, jnp.int32), cellv)
            m0_v[sl] = plsc.load_gather(plane_v, [gidx])
        pltpu.sync_copy(mos_h.at[b, 1], plane_v)
        for j in range(ppw // 16):
            sl = pl.ds(j * 16, 16)
            cellv = cell_v[sl]
            gidx = jnp.where(cellv < 0, jnp.full((16,), 0, jnp.int32), cellv)
            m1_v[sl] = plsc.load_gather(plane_v, [gidx])
        pltpu.sync_copy(cell_v, cell_o.at[r])
        pltpu.sync_copy(lab_v, lab_o.at[r])
        pltpu.sync_copy(m0_v, m0_o.at[r])
        pltpu.sync_copy(m1_v, m1_o.at[r])

    f = pl.kernel(
        _sc_body,
        out_type=(
            jax.ShapeDtypeStruct((nw, ppw), jnp.int32),
            jax.ShapeDtypeStruct((nw, ppw), jnp.int32),
            jax.ShapeDtypeStruct((nw, ppw), jnp.float32),
            jax.ShapeDtypeStruct((nw, ppw), jnp.float32),
        ),
        mesh=mesh,
        compiler_params=pltpu.CompilerParams(
            needs_layout_passes=False, use_tc_tiling_on_sc=False),
        scratch_types=(
            pltpu.VMEM((N,), jnp.float32),
            pltpu.VMEM((N,), jnp.float32),
            pltpu.VMEM((ppw,), jnp.float32),
            pltpu.VMEM((ppw,), jnp.float32),
            pltpu.VMEM((ppw,), jnp.float32),
            pltpu.VMEM((ppw,), jnp.int32),
            pltpu.VMEM((ppw,), jnp.int32),
            pltpu.VMEM((ppw,), jnp.int32),
            pltpu.VMEM((ppw,), jnp.int32),
            pltpu.VMEM((ppw,), jnp.float32),
            pltpu.VMEM((ppw,), jnp.float32),
            pltpu.VMEM((_GG,), jnp.float32),
            pltpu.SemaphoreType.DMA,
        ),
    )
    cell_o, lab_o, m0_o, m1_o = f(pjT, dx, dy, ef, ix, nf, mos2)
    return (cell_o.reshape(B, 1, N), lab_o.reshape(B, 1, N),
            m0_o.reshape(B, 1, N), m1_o.reshape(B, 1, N))


def kernel(p_i, mos, p_j, error_p_i_flow, nearest_flow):
    B, N, _ = p_i.shape
    piT = p_i.transpose(0, 2, 1)
    pjT = p_j.transpose(0, 2, 1)

    dx, ix, dy = _knn_call(piT, pjT)

    cell, lab, m0, m1 = _sc_stage(
        pjT, dx, dy, error_p_i_flow, ix, nearest_flow[..., 0],
        mos.reshape(B, 2, _GG))

    loss = _winner_call(cell, lab, m0, m1)
    return loss[0, 0]


# speedup vs baseline: 1.6871x; 1.1742x over previous
import jax
import jax.numpy as jnp
from jax import lax
from jax.experimental import pallas as pl
from jax.experimental.pallas import tpu as pltpu
from jax.experimental.pallas import tpu_sc as plsc

_G = 320
_GG = _G * _G
_X_MIN = -35.0
_Y_MIN = -35.0
_CELL = abs(2.0 * _X_MIN / _G)
_TI = 512

_DN_COL = (((1,), (1,)), ((), ()))
_DN_ROW = (((0,), (0,)), ((), ()))


def _eye():
    return (lax.broadcasted_iota(jnp.int32, (_TI, _TI), 0)
            == lax.broadcasted_iota(jnp.int32, (_TI, _TI), 1)
            ).astype(jnp.float32)


def _to_col(eye, seg):
    return lax.dot_general(eye, seg, _DN_COL,
                           preferred_element_type=jnp.float32,
                           precision=lax.Precision.HIGHEST)


def _to_row(col, eye):
    return lax.dot_general(col, eye, _DN_ROW,
                           preferred_element_type=jnp.float32,
                           precision=lax.Precision.HIGHEST)


def _knn_body(piT, pjT, dxo, ixo, dyo):
    N = pjT.shape[2]
    t = pl.program_id(1)
    off = pl.multiple_of(t * _TI, _TI)
    eye = _eye()
    isl = pl.ds(off, _TI)
    xit = _to_col(eye, piT[0, pl.ds(0, 1), isl])
    yit = _to_col(eye, piT[0, pl.ds(1, 1), isl])
    zit = _to_col(eye, piT[0, pl.ds(2, 1), isl])
    d = jnp.abs(xit - pjT[0, pl.ds(0, 1), :])
    d = d + jnp.abs(yit - pjT[0, pl.ds(1, 1), :])
    d = d + jnp.abs(zit - pjT[0, pl.ds(2, 1), :])
    rmin = jnp.min(d, axis=1, keepdims=True)
    jj = lax.broadcasted_iota(jnp.int32, (_TI, N), 1)
    amin = jnp.min(jnp.where(d == rmin, jj, N), axis=1, keepdims=True)
    dxo[0, :, isl] = _to_row(rmin, eye)
    ixo[0, :, isl] = _to_row(amin.astype(jnp.float32), eye).astype(jnp.int32)
    cmin = jnp.min(d, axis=0, keepdims=True)

    @pl.when(t == 0)
    def _():
        dyo[0] = cmin

    @pl.when(t != 0)
    def _():
        dyo[0] = jnp.minimum(dyo[0], cmin)


def _knn_call(piT, pjT):
    B, _, N = pjT.shape
    nt = N // _TI
    coords = pl.BlockSpec((1, 3, N), lambda b, t: (b, 0, 0))
    row = pl.BlockSpec((1, 1, N), lambda b, t: (b, 0, 0))
    return pl.pallas_call(
        _knn_body,
        grid=(B, nt),
        in_specs=[coords, coords],
        out_specs=[row, row, row],
        out_shape=[
            jax.ShapeDtypeStruct((B, 1, N), jnp.float32),
            jax.ShapeDtypeStruct((B, 1, N), jnp.int32),
            jax.ShapeDtypeStruct((B, 1, N), jnp.float32),
        ],
        compiler_params=pltpu.CompilerParams(
            dimension_semantics=("arbitrary", "arbitrary"),
        ),
    )(piT, pjT)


def _winner_body(cell, lab, m0, m1, out, psum, pcnt):
    N = cell.shape[2]
    b = pl.program_id(0)
    t = pl.program_id(1)
    nb = pl.num_programs(0)
    nt = pl.num_programs(1)
    off = pl.multiple_of(t * _TI, _TI)
    isl = pl.ds(off, _TI)
    eye = _eye()

    @pl.when((b == 0) & (t == 0))
    def _():
        psum[...] = jnp.zeros_like(psum)
        pcnt[...] = jnp.zeros_like(pcnt)

    rowcf = cell[0].astype(jnp.float32)
    cif = _to_col(eye, cell[0, :, isl].astype(jnp.float32))
    jj = lax.broadcasted_iota(jnp.int32, (_TI, N), 1)
    ii = lax.broadcasted_iota(jnp.int32, (_TI, 1), 0) + t * _TI
    hit = (cif == rowcf) & (jj > ii)
    later = jnp.any(hit, axis=1, keepdims=True)
    win = (cif != -1.0) & jnp.logical_not(later)
    m0i = _to_col(eye, m0[0, :, isl])
    m1i = _to_col(eye, m1[0, :, isl])
    labi = _to_col(eye, lab[0, :, isl].astype(jnp.float32))
    mx = jnp.maximum(m0i, m1i)
    lse = jnp.log(jnp.exp(m0i - mx) + jnp.exp(m1i - mx)) + mx
    sel = jnp.where(labi == 1.0, m1i, m0i)
    psum[...] = psum[...] + jnp.sum(jnp.where(win, sel - lse, 0.0))
    pcnt[...] = pcnt[...] + jnp.sum(win.astype(jnp.float32))

    @pl.when((b == nb - 1) & (t == nt - 1))
    def _():
        out[...] = -(psum[...] / pcnt[...])


def _winner_call(cell, lab, m0, m1):
    B, _, N = cell.shape
    nt = N // _TI
    row = pl.BlockSpec((1, 1, N), lambda b, t: (b, 0, 0))
    return pl.pallas_call(
        _winner_body,
        grid=(B, nt),
        in_specs=[row, row, row, row],
        out_specs=pl.BlockSpec((1, 1), lambda b, t: (0, 0)),
        out_shape=jax.ShapeDtypeStruct((1, 1), jnp.float32),
        scratch_shapes=[
            pltpu.VMEM((1, 1), jnp.float32),
            pltpu.VMEM((1, 1), jnp.float32),
        ],
        compiler_params=pltpu.CompilerParams(
            dimension_semantics=("arbitrary", "arbitrary"),
        ),
    )(cell, lab, m0, m1)


def _sc_stage(pjT, dx, dy, ef, ix, nf, mos2):
    B, _, N = pjT.shape
    npt = B * N
    nw = 32
    ppw = npt // nw
    wpb = N // ppw

    mesh = plsc.VectorSubcoreMesh(core_axis_name="c", subcore_axis_name="s")

    def _sc_body(pjT_h, dx_h, dy_h, ef_h, ix_h, nf_h, mos_h,
                 cell_o, lab_o, m0_o, m1_o,
                 pjx_v, pjy_v, dx_v, dy_v, ef_v, ix_v, nf_v,
                 cell_v, lab_v, m0_v, m1_v, plane_v, sem):
        c = lax.axis_index("c")
        s = lax.axis_index("s")
        b = (c * 16 + s) // wpb
        off = ((c * 16 + s) % wpb) * ppw
        r = c * 16 + s
        pltpu.sync_copy(pjT_h.at[b, 0], pjx_v)
        pltpu.sync_copy(pjT_h.at[b, 1], pjy_v)
        pltpu.sync_copy(dx_h.at[b, 0, pl.ds(off, ppw)], dx_v)
        pltpu.sync_copy(dy_h.at[b, 0, pl.ds(off, ppw)], dy_v)
        pltpu.sync_copy(ef_h.at[b, pl.ds(off, ppw)], ef_v)
        pltpu.sync_copy(ix_h.at[b, 0, pl.ds(off, ppw)], ix_v)
        pltpu.sync_copy(nf_h.at[b, pl.ds(off, ppw)], nf_v)
        one16 = jnp.full((16,), 1, jnp.int32)
        zero16 = jnp.full((16,), 0, jnp.int32)
        neg16 = jnp.full((16,), -1, jnp.int32)
        for j in range(ppw // 16):
            sl = pl.ds(j * 16, 16)
            err = (dx_v[sl] + dy_v[sl]) / 2.0
            dyn = ef_v[sl] > err
            idxv = jnp.where(dyn, nf_v[sl], ix_v[sl])
            labv = jnp.where(dyn, one16, zero16)
            xjv = plsc.load_gather(pjx_v, [idxv])
            yjv = plsc.load_gather(pjy_v, [idxv])
            cxi = ((xjv - _X_MIN) / _CELL).astype(jnp.int32)
            cyi = ((yjv - _Y_MIN) / _CELL).astype(jnp.int32)
            okv = (cxi >= 0) & (cxi < _G) & (cyi >= 0) & (cyi < _G)
            cellv = jnp.where(okv, cxi * _G + cyi, neg16)
            cell_v[sl] = cellv
            lab_v[sl] = labv
        pltpu.sync_copy(mos_h.at[b, 0], plane_v)
        for j in range(ppw // 16):
            sl = pl.ds(j * 16, 16)
            cellv = cell_v[sl]
            gidx = jnp.where(cellv < 0, jnp.full((16,), 0, jnp.int32), cellv)
            m0_v[sl] = plsc.load_gather(plane_v, [gidx])
        pltpu.sync_copy(mos_h.at[b, 1], plane_v)
        for j in range(ppw // 16):
            sl = pl.ds(j * 16, 16)
            cellv = cell_v[sl]
            gidx = jnp.where(cellv < 0, jnp.full((16,), 0, jnp.int32), cellv)
            m1_v[sl] = plsc.load_gather(plane_v, [gidx])
        pltpu.sync_copy(cell_v, cell_o.at[r])
        pltpu.sync_copy(lab_v, lab_o.at[r])
        pltpu.sync_copy(m0_v, m0_o.at[r])
        pltpu.sync_copy(m1_v, m1_o.at[r])

    f = pl.kernel(
        _sc_body,
        out_type=(
            jax.ShapeDtypeStruct((nw, ppw), jnp.int32),
            jax.ShapeDtypeStruct((nw, ppw), jnp.int32),
            jax.ShapeDtypeStruct((nw, ppw), jnp.float32),
            jax.ShapeDtypeStruct((nw, ppw), jnp.float32),
        ),
        mesh=mesh,
        compiler_params=pltpu.CompilerParams(
            needs_layout_passes=False, use_tc_tiling_on_sc=False),
        scratch_types=(
            pltpu.VMEM((N,), jnp.float32),
            pltpu.VMEM((N,), jnp.float32),
            pltpu.VMEM((ppw,), jnp.float32),
            pltpu.VMEM((ppw,), jnp.float32),
            pltpu.VMEM((ppw,), jnp.float32),
            pltpu.VMEM((ppw,), jnp.int32),
            pltpu.VMEM((ppw,), jnp.int32),
            pltpu.VMEM((ppw,), jnp.int32),
            pltpu.VMEM((ppw,), jnp.int32),
            pltpu.VMEM((ppw,), jnp.float32),
            pltpu.VMEM((ppw,), jnp.float32),
            pltpu.VMEM((_GG,), jnp.float32),
            pltpu.SemaphoreType.DMA,
        ),
    )
    cell_o, lab_o, m0_o, m1_o = f(pjT, dx, dy, ef, ix, nf, mos2)
    return (cell_o.reshape(B, 1, N), lab_o.reshape(B, 1, N),
            m0_o.reshape(B, 1, N), m1_o.reshape(B, 1, N))


def kernel(p_i, mos, p_j, error_p_i_flow, nearest_flow):
    B, N, _ = p_i.shape
    piT = p_i.transpose(0, 2, 1)
    pjT = p_j.transpose(0, 2, 1)

    dx, ix, dy = _knn_call(piT, pjT)

    cell, lab, m0, m1 = _sc_stage(
        pjT, dx, dy, error_p_i_flow, ix, nearest_flow[..., 0],
        mos.reshape(B, 2, _GG))

    loss = _winner_call(cell, lab, m0, m1)
    return loss[0, 0]


# speedup vs baseline: 2.0789x; 1.2323x over previous
import jax
import jax.numpy as jnp
from jax import lax
from jax.experimental import pallas as pl
from jax.experimental.pallas import tpu as pltpu
from jax.experimental.pallas import tpu_sc as plsc

_G = 320
_GG = _G * _G
_X_MIN = -35.0
_Y_MIN = -35.0
_CELL = abs(2.0 * _X_MIN / _G)
_TI = 512

_DN_COL = (((1,), (1,)), ((), ()))
_DN_ROW = (((0,), (0,)), ((), ()))


def _eye():
    return (lax.broadcasted_iota(jnp.int32, (_TI, _TI), 0)
            == lax.broadcasted_iota(jnp.int32, (_TI, _TI), 1)
            ).astype(jnp.float32)


def _to_col(eye, seg):
    return lax.dot_general(eye, seg, _DN_COL,
                           preferred_element_type=jnp.float32,
                           precision=lax.Precision.HIGHEST)


def _to_row(col, eye):
    return lax.dot_general(col, eye, _DN_ROW,
                           preferred_element_type=jnp.float32,
                           precision=lax.Precision.HIGHEST)


def _knn_body(piT, pjT, dxo, ixo, dyo):
    N = pjT.shape[2]
    t = pl.program_id(1)
    off = pl.multiple_of(t * _TI, _TI)
    eye = _eye()
    isl = pl.ds(off, _TI)
    xit = _to_col(eye, piT[0, pl.ds(0, 1), isl])
    yit = _to_col(eye, piT[0, pl.ds(1, 1), isl])
    zit = _to_col(eye, piT[0, pl.ds(2, 1), isl])
    d = jnp.abs(xit - pjT[0, pl.ds(0, 1), :])
    d = d + jnp.abs(yit - pjT[0, pl.ds(1, 1), :])
    d = d + jnp.abs(zit - pjT[0, pl.ds(2, 1), :])
    rmin = jnp.min(d, axis=1, keepdims=True)
    jj = lax.broadcasted_iota(jnp.int32, (_TI, N), 1)
    amin = jnp.min(jnp.where(d == rmin, jj, N), axis=1, keepdims=True)
    dxo[0, :, isl] = _to_row(rmin, eye)
    ixo[0, :, isl] = _to_row(amin.astype(jnp.float32), eye).astype(jnp.int32)
    cmin = jnp.min(d, axis=0, keepdims=True)

    @pl.when(t == 0)
    def _():
        dyo[0] = cmin

    @pl.when(t != 0)
    def _():
        dyo[0] = jnp.minimum(dyo[0], cmin)


def _knn_call(piT, pjT):
    B, _, N = pjT.shape
    nt = N // _TI
    coords = pl.BlockSpec((1, 3, N), lambda b, t: (b, 0, 0))
    row = pl.BlockSpec((1, 1, N), lambda b, t: (b, 0, 0))
    return pl.pallas_call(
        _knn_body,
        grid=(B, nt),
        in_specs=[coords, coords],
        out_specs=[row, row, row],
        out_shape=[
            jax.ShapeDtypeStruct((B, 1, N), jnp.float32),
            jax.ShapeDtypeStruct((B, 1, N), jnp.int32),
            jax.ShapeDtypeStruct((B, 1, N), jnp.float32),
        ],
        compiler_params=pltpu.CompilerParams(
            dimension_semantics=("arbitrary", "arbitrary"),
        ),
    )(piT, pjT)


def _ce_body(grid, mos, out, psum, pcnt):
    b = pl.program_id(0)
    nb = pl.num_programs(0)

    @pl.when(b == 0)
    def _():
        psum[...] = jnp.zeros_like(psum)
        pcnt[...] = jnp.zeros_like(pcnt)

    g = grid[0]
    m0 = mos[0, pl.ds(0, 1), :]
    m1 = mos[0, pl.ds(1, 1), :]
    valid = g >= 0
    mx = jnp.maximum(m0, m1)
    lse = jnp.log(jnp.exp(m0 - mx) + jnp.exp(m1 - mx)) + mx
    sel = jnp.where(g == 1, m1, m0)
    psum[...] = psum[...] + jnp.sum(jnp.where(valid, sel - lse, 0.0))
    pcnt[...] = pcnt[...] + jnp.sum(valid.astype(jnp.float32))

    @pl.when(b == nb - 1)
    def _():
        out[...] = -(psum[...] / pcnt[...])


def _ce_call(grid, mos2):
    B, _, GG = grid.shape
    return pl.pallas_call(
        _ce_body,
        grid=(B,),
        in_specs=[
            pl.BlockSpec((1, 1, GG), lambda b: (b, 0, 0)),
            pl.BlockSpec((1, 2, GG), lambda b: (b, 0, 0)),
        ],
        out_specs=pl.BlockSpec((1, 1), lambda b: (0, 0)),
        out_shape=jax.ShapeDtypeStruct((1, 1), jnp.float32),
        scratch_shapes=[
            pltpu.VMEM((1, 1), jnp.float32),
            pltpu.VMEM((1, 1), jnp.float32),
        ],
        compiler_params=pltpu.CompilerParams(
            dimension_semantics=("arbitrary",),
        ),
    )(grid, mos2)


def _sc_stage(pjT, dx, dy, ef, ix, nf, ginit):
    B, _, N = pjT.shape
    npt = B * N
    nw = 32
    ppw = npt // nw
    wpb = N // ppw

    mesh = plsc.VectorSubcoreMesh(core_axis_name="c", subcore_axis_name="s")

    def _sc_body(pjT_h, dx_h, dy_h, ef_h, ix_h, nf_h, ginit_h,
                 grid_o,
                 pjx_v, pjy_v, dx_v, dy_v, ef_v, ix_v, nf_v,
                 cell_v, lab_v, cell8_v, lab8_v, grid_v,
                 cell_sh, lab_sh, sem):
        c = lax.axis_index("c")
        s = lax.axis_index("s")
        b = (c * 16 + s) // wpb
        off = ((c * 16 + s) % wpb) * ppw
        pltpu.sync_copy(pjT_h.at[b, 0], pjx_v)
        pltpu.sync_copy(pjT_h.at[b, 1], pjy_v)
        pltpu.sync_copy(dx_h.at[b, 0, pl.ds(off, ppw)], dx_v)
        pltpu.sync_copy(dy_h.at[b, 0, pl.ds(off, ppw)], dy_v)
        pltpu.sync_copy(ef_h.at[b, pl.ds(off, ppw)], ef_v)
        pltpu.sync_copy(ix_h.at[b, 0, pl.ds(off, ppw)], ix_v)
        pltpu.sync_copy(nf_h.at[b, pl.ds(off, ppw)], nf_v)
        one16 = jnp.full((16,), 1, jnp.int32)
        zero16 = jnp.full((16,), 0, jnp.int32)
        neg16 = jnp.full((16,), -1, jnp.int32)
        for j in range(ppw // 16):
            sl = pl.ds(j * 16, 16)
            err = (dx_v[sl] + dy_v[sl]) / 2.0
            dyn = ef_v[sl] > err
            idxv = jnp.where(dyn, nf_v[sl], ix_v[sl])
            labv = jnp.where(dyn, one16, zero16)
            xjv = plsc.load_gather(pjx_v, [idxv])
            yjv = plsc.load_gather(pjy_v, [idxv])
            cxi = ((xjv - _X_MIN) / _CELL).astype(jnp.int32)
            cyi = ((yjv - _Y_MIN) / _CELL).astype(jnp.int32)
            okv = (cxi >= 0) & (cxi < _G) & (cyi >= 0) & (cyi < _G)
            cellv = jnp.where(okv, cxi * _G + cyi, neg16)
            cell_v[sl] = cellv
            lab_v[sl] = labv
        pltpu.sync_copy(cell_v, cell_sh.at[s])
        pltpu.sync_copy(lab_v, lab_sh.at[s])
        is_scatter = (s == 0) | (s == wpb)

        @pl.when(is_scatter)
        def _():
            pltpu.sync_copy(ginit_h, grid_v)
        plsc.subcore_barrier()

        @pl.when(is_scatter)
        def _():
            pltpu.sync_copy(cell_sh.at[pl.ds(s, wpb)], cell8_v)
            pltpu.sync_copy(lab_sh.at[pl.ds(s, wpb)], lab8_v)
            for q in range(wpb):
                for k in range(ppw // 16):
                    sl = pl.ds(k * 16, 16)
                    cv = cell8_v[q, sl]
                    lv = lab8_v[q, sl]
                    plsc.store_scatter(grid_v, [cv], lv, mask=cv >= 0)
            pltpu.sync_copy(grid_v, grid_o.at[b])

    f = pl.kernel(
        _sc_body,
        out_type=jax.ShapeDtypeStruct((B, _GG), jnp.int32),
        mesh=mesh,
        compiler_params=pltpu.CompilerParams(
            needs_layout_passes=False, use_tc_tiling_on_sc=False),
        scratch_types=(
            pltpu.VMEM((N,), jnp.float32),
            pltpu.VMEM((N,), jnp.float32),
            pltpu.VMEM((ppw,), jnp.float32),
            pltpu.VMEM((ppw,), jnp.float32),
            pltpu.VMEM((ppw,), jnp.float32),
            pltpu.VMEM((ppw,), jnp.int32),
            pltpu.VMEM((ppw,), jnp.int32),
            pltpu.VMEM((ppw,), jnp.int32),
            pltpu.VMEM((ppw,), jnp.int32),
            pltpu.VMEM((wpb, ppw), jnp.int32),
            pltpu.VMEM((wpb, ppw), jnp.int32),
            pltpu.VMEM((_GG,), jnp.int32),
            pltpu.VMEM_SHARED((16, ppw), jnp.int32),
            pltpu.VMEM_SHARED((16, ppw), jnp.int32),
            pltpu.SemaphoreType.DMA,
        ),
    )
    grid_o = f(pjT, dx, dy, ef, ix, nf, ginit)
    return grid_o.reshape(B, 1, _GG)


def kernel(p_i, mos, p_j, error_p_i_flow, nearest_flow):
    B, N, _ = p_i.shape
    piT = p_i.transpose(0, 2, 1)
    pjT = p_j.transpose(0, 2, 1)

    dx, ix, dy = _knn_call(piT, pjT)

    ginit = jnp.full((_GG,), -1, jnp.int32)
    grid = _sc_stage(pjT, dx, dy, error_p_i_flow, ix,
                     nearest_flow[..., 0], ginit)

    loss = _ce_call(grid, mos.reshape(B, 2, _GG))
    return loss[0, 0]
